# Initial kernel scaffold; baseline (speedup 1.0000x reference)
import jax
import jax.numpy as jnp
from jax.experimental import pallas as pl


def kernel(x):
    raise NotImplementedError("write your pallas kernel here")


# speedup vs baseline: 400.7371x; 400.7371x over previous
import jax
import jax.numpy as jnp
from jax.experimental import pallas as pl

_T = 8


def _encode_block(x_ref, out_ref):
    x = x_ref[...]
    t = (jax.nn.sigmoid(x) * (_T - 1)).astype(jnp.int32)
    for ti in range(_T):
        out_ref[0, ti] = (t[0] == ti).astype(jnp.float32)


def kernel(x):
    B, D1, D2 = x.shape
    BD1 = 256
    grid = (B, D1 // BD1)
    return pl.pallas_call(
        _encode_block,
        grid=grid,
        in_specs=[pl.BlockSpec((1, BD1, D2), lambda b, i: (b, i, 0))],
        out_specs=pl.BlockSpec((1, _T, BD1, D2), lambda b, i: (b, 0, i, 0)),
        out_shape=jax.ShapeDtypeStruct((B, _T, D1, D2), jnp.float32),
    )(x)
